# baseline (speedup 1.0000x reference)
import jax
import jax.numpy as jnp
from jax.experimental import pallas as pl
from jax.experimental.pallas import tpu as pltpu
from jax.experimental.pallas import tpu_sc as plsc

_B, _S, _D, _H, _O, _SKP, _E = 8, 512, 256, 256, 256, 256, 8
_TS = 64
_NCHUNK = _S // _TS
_L = 16
F32 = jnp.float32
BF16 = jnp.bfloat16
_NEG = -1e30


def _sc_gating_kernel(spk_hbm, wg_hbm, bg_hbm, gates_hbm, loss_hbm,
                      spk_v, wg_v, bg_v, gates_v, loss_v):
    cid = jax.lax.axis_index("c")
    sid = jax.lax.axis_index("s")

    pltpu.sync_copy(spk_hbm, spk_v)
    pltpu.sync_copy(wg_hbm, wg_v)
    pltpu.sync_copy(bg_hbm, bg_v)
    lane = jax.lax.iota(jnp.int32, _L)

    def perm(v, idx):
        return jax.lax.gather(
            v, idx[:, None],
            dimension_numbers=jax.lax.GatherDimensionNumbers(
                offset_dims=(), collapsed_slice_dims=(0,),
                start_index_map=(0,)),
            slice_sizes=(1,),
            mode=jax.lax.GatherScatterMode.PROMISE_IN_BOUNDS)

    def allred(v, op):
        for s in (1, 2, 4, 8):
            v = op(v, perm(v, jnp.bitwise_xor(lane, s)))
        return v

    bgv = bg_v[...]
    imp = jnp.zeros((_L,), F32)
    load = jnp.zeros((_L,), F32)
    for b in range(_B):
        chunks = [spk_v[b, pl.ds(k * _L, _L)] for k in range(_SKP // _L)]
        logits = jnp.full((_L,), _NEG, F32)
        for e in range(_E):
            acc = chunks[0] * wg_v[e, pl.ds(0, _L)]
            for k in range(1, _SKP // _L):
                acc = acc + chunks[k] * wg_v[e, pl.ds(k * _L, _L)]
            logits = jnp.where(lane == e, allred(acc, jnp.add), logits)
        logits = logits + bgv
        m1 = allred(logits, jnp.maximum)
        i1 = allred(jnp.where(logits == m1, lane, _L), jnp.minimum)
        masked = jnp.where(lane == i1, _NEG, logits)
        m2 = allred(masked, jnp.maximum)
        i2 = allred(jnp.where(masked == m2, lane, _L), jnp.minimum)
        e2 = jnp.exp(m2 - m1)
        denom = 1.0 + e2
        g = jnp.where(lane == i1, 1.0 / denom,
                      jnp.where(lane == i2, e2 / denom, 0.0))
        gates_v[b, :] = g
        imp = imp + g
        load = load + jnp.where(g > 0.0, 1.0, 0.0)

    def cv2(v):
        mean = allred(v, jnp.add) / _E
        d = jnp.where(lane < _E, v - mean, 0.0)
        var = allred(d * d, jnp.add) / (_E - 1)
        return var / (mean * mean + 1e-10)

    loss_v[...] = (cv2(imp) + cv2(load)) * 0.01

    @pl.when((cid == 0) & (sid == 0))
    def _():
        pltpu.sync_copy(gates_v, gates_hbm)
        pltpu.sync_copy(loss_v, loss_hbm)


def _sc_gating(spk, Wg, bg):
    bg_pad = jnp.zeros((_L,), F32).at[:_E].set(bg)
    gates_pad, loss_pad = pl.kernel(
        _sc_gating_kernel,
        out_type=(
            jax.ShapeDtypeStruct((_B, _L), F32),
            jax.ShapeDtypeStruct((_L,), F32),
        ),
        mesh=plsc.VectorSubcoreMesh(core_axis_name="c", subcore_axis_name="s"),
        scratch_types=[
            pltpu.VMEM((_B, _SKP), F32),
            pltpu.VMEM((_E, _SKP), F32),
            pltpu.VMEM((_L,), F32),
            pltpu.VMEM((_B, _L), F32),
            pltpu.VMEM((_L,), F32),
        ],
    )(spk, Wg, bg_pad)
    return gates_pad[:, :_E], loss_pad[0]


def _lstm_kernel(x2_ref, wih_ref, whh_ref, b_ref, gcol_ref, wfc_ref, bfc_ref,
                 out_ref, xw_ref, comb_ref, h_ref, c_ref):
    t0 = pl.program_id(0)

    xc = x2_ref[...].astype(BF16)
    for e in range(_E):
        xw_ref[e] = (
            jnp.dot(xc, wih_ref[e], preferred_element_type=F32) + b_ref[e]
        )

    @pl.when(t0 == 0)
    def _():
        h_ref[...] = jnp.zeros_like(h_ref)
        c_ref[...] = jnp.zeros_like(c_ref)

    ge = [gcol_ref[e] for e in range(_E)]

    h0 = tuple(h_ref[e * _B:(e + 1) * _B, :] for e in range(_E))
    c0 = tuple(c_ref[e * _B:(e + 1) * _B, :] for e in range(_E))

    def step(t, carry):
        hs, cs = carry
        sl = pl.ds(t * _B, _B)
        comb = None
        new_h, new_c = [], []
        for e in range(_E):
            z = xw_ref[e, sl, :] + jnp.dot(
                hs[e].astype(BF16), whh_ref[e], preferred_element_type=F32)
            i = jax.nn.sigmoid(z[:, :_H])
            f = jax.nn.sigmoid(z[:, _H:2 * _H])
            g = jnp.tanh(z[:, 2 * _H:3 * _H])
            o = jax.nn.sigmoid(z[:, 3 * _H:])
            ce = f * cs[e] + i * g
            he = o * jnp.tanh(ce)
            new_h.append(he)
            new_c.append(ce)
            contrib = ge[e] * he
            comb = contrib if comb is None else comb + contrib
        comb_ref[sl, :] = comb
        return (tuple(new_h), tuple(new_c))

    hs_fin, cs_fin = jax.lax.fori_loop(0, _TS, step, (h0, c0), unroll=8)
    for e in range(_E):
        h_ref[e * _B:(e + 1) * _B, :] = hs_fin[e]
        c_ref[e * _B:(e + 1) * _B, :] = cs_fin[e]

    out_ref[...] = (
        jnp.dot(comb_ref[...].astype(BF16), wfc_ref[...],
                preferred_element_type=F32)
        + bfc_ref[...]
    )


def kernel(x, spk, Wg, bg, W_ih, W_hh, b_ih, b_hh, W_fc, b_fc):
    x2 = jnp.swapaxes(x, 0, 1).reshape(_S * _B, _D)
    W_ihT = jnp.swapaxes(W_ih, 1, 2).astype(BF16)
    W_hhT = jnp.swapaxes(W_hh, 1, 2).astype(BF16)
    b = (b_ih + b_hh).reshape(_E, 1, 4 * _H)
    W_fcT = W_fc.T.astype(BF16)
    bfc2 = b_fc.reshape(1, _O)

    gates, loss = _sc_gating(spk, Wg, bg)

    gcol = gates.T.reshape(_E, _B, 1)

    out2 = pl.pallas_call(
        _lstm_kernel,
        grid=(_NCHUNK,),
        in_specs=[
            pl.BlockSpec((_TS * _B, _D), lambda t: (t, 0)),
            pl.BlockSpec((_E, _D, 4 * _H), lambda t: (0, 0, 0)),
            pl.BlockSpec((_E, _H, 4 * _H), lambda t: (0, 0, 0)),
            pl.BlockSpec((_E, 1, 4 * _H), lambda t: (0, 0, 0)),
            pl.BlockSpec((_E, _B, 1), lambda t: (0, 0, 0)),
            pl.BlockSpec((_H, _O), lambda t: (0, 0)),
            pl.BlockSpec((1, _O), lambda t: (0, 0)),
        ],
        out_specs=pl.BlockSpec((_TS * _B, _O), lambda t: (t, 0)),
        out_shape=jax.ShapeDtypeStruct((_S * _B, _O), F32),
        scratch_shapes=[
            pltpu.VMEM((_E, _TS * _B, 4 * _H), F32),
            pltpu.VMEM((_TS * _B, _H), F32),
            pltpu.VMEM((_E * _B, _H), F32),
            pltpu.VMEM((_E * _B, _H), F32),
        ],
    )(x2, W_ihT, W_hhT, b, gcol, W_fcT, bfc2)

    out = out2.reshape(_S, _B, _O).swapaxes(0, 1)
    return (out, loss)
